# baseline (speedup 1.0000x reference)
import functools

import jax
import jax.numpy as jnp
from jax import lax
from jax.experimental import pallas as pl
from jax.experimental.pallas import tpu as pltpu
from jax.experimental.pallas import tpu_sc as plsc

N = 1600000
NUM_SEGMENTS = 4096
NW = 32
CHUNKS = 5
CHUNK_ROWS = N // CHUNKS
ROWS_PER_W = CHUNK_ROWS // NW
TILE = 2000
STEPS = ROWS_PER_W // TILE
VECS = TILE // 16

_NEG_INF = float("-inf")


def _sc_stage(t0, t1, t2, vcol, batch_idx, wvec):
    mesh = plsc.VectorSubcoreMesh(core_axis_name="c", subcore_axis_name="s")

    @functools.partial(
        pl.kernel,
        mesh=mesh,
        compiler_params=pltpu.CompilerParams(needs_layout_passes=False),
        out_type=jax.ShapeDtypeStruct((NW, NUM_SEGMENTS), jnp.float32),
        scratch_types=[
            pltpu.VMEM((TILE * 3,), jnp.float32),
            pltpu.VMEM((TILE * 3,), jnp.float32),
            pltpu.VMEM((TILE,), jnp.float32),
            pltpu.VMEM((TILE,), jnp.float32),
            pltpu.VMEM((TILE,), jnp.int32),
            pltpu.VMEM((TILE,), jnp.int32),
            pltpu.VMEM((16, NUM_SEGMENTS), jnp.float32),
            pltpu.VMEM((NUM_SEGMENTS,), jnp.float32),
            pltpu.VMEM((80,), jnp.float32),
            pltpu.SemaphoreType.DMA,
            pltpu.SemaphoreType.DMA,
        ],
    )
    def k(t0_hbm, t1_hbm, t2_hbm, votes_hbm, idx_hbm, wv_hbm, out_hbm,
          tv0, tv1, vv0, vv1, iv0, iv1, acc, red, wv_v, sem0, sem1):
        wid = lax.axis_index("s") * 2 + lax.axis_index("c")
        sems = (sem0, sem1)
        tvs, vvs, ivs = (tv0, tv1), (vv0, vv1), (iv0, iv1)

        lane = lax.iota(jnp.int32, 16)
        ninf = jnp.full((16,), _NEG_INF, jnp.float32)

        handles = [None, None]

        def fire(s):
            b = s % 2
            row0 = wid * ROWS_PER_W + s * TILE
            hs = []
            for j, col in enumerate((t0_hbm, t1_hbm, t2_hbm)):
                hs.append(pltpu.async_copy(
                    col.at[pl.ds(row0, TILE)],
                    tvs[b].at[pl.ds(j * TILE, TILE)], sems[b]))
            hs.append(pltpu.async_copy(
                votes_hbm.at[pl.ds(row0, TILE)], vvs[b], sems[b]))
            hs.append(pltpu.async_copy(
                idx_hbm.at[pl.ds(row0, TILE)], ivs[b], sems[b]))
            handles[b] = hs

        fire(0)

        pltpu.sync_copy(wv_hbm, wv_v)
        w0 = wv_v[pl.ds(0, 16)]
        w1 = wv_v[pl.ds(16, 16)]
        w2 = wv_v[pl.ds(32, 16)]
        w3 = wv_v[pl.ds(48, 16)]
        w4 = wv_v[pl.ds(64, 16)]

        def init_body(j, _):
            for r in range(16):
                acc[r, pl.ds(j * 16, 16)] = ninf
            return 0
        lax.fori_loop(0, NUM_SEGMENTS // 16, init_body, 0)

        for s in range(STEPS):
            if s + 1 < STEPS:
                fire(s + 1)
            b = s % 2
            for h in handles[b]:
                h.wait()

            def vec_body(v, _):
                c0 = tvs[b][pl.ds(v * 16, 16)]
                c1 = tvs[b][pl.ds(TILE + v * 16, 16)]
                c2 = tvs[b][pl.ds(2 * TILE + v * 16, 16)]
                vvv = vvs[b][pl.ds(v * 16, 16)]
                ivv = ivs[b][pl.ds(v * 16, 16)]
                x = c0 * w0 + c1 * w1 + c2 * w2 + vvv * w3 + w4
                g = plsc.load_gather(acc, [lane, ivv])
                plsc.store_scatter(acc, [lane, ivv], jnp.maximum(g, x))
                return 0
            lax.fori_loop(0, VECS, vec_body, 0)

        def red_body(j, _):
            m = acc[0, pl.ds(j * 16, 16)]
            for r in range(1, 16):
                m = jnp.maximum(m, acc[r, pl.ds(j * 16, 16)])
            red[pl.ds(j * 16, 16)] = m
            return 0
        lax.fori_loop(0, NUM_SEGMENTS // 16, red_body, 0)

        pltpu.sync_copy(red, out_hbm.at[wid])

    return k(t0, t1, t2, vcol, batch_idx, wvec)


def _tc_reduce(partials):
    def body(*refs):
        p_refs, o_ref = refs[:-1], refs[-1]
        m = p_refs[0][...]
        for p in p_refs[1:]:
            m = jnp.maximum(m, p[...])
        o_ref[...] = jnp.max(m, axis=0, keepdims=True)

    return pl.pallas_call(
        body,
        out_shape=jax.ShapeDtypeStruct((1, NUM_SEGMENTS), jnp.float32),
    )(*partials)


def kernel(trunk, votes, batch_idx, W, b):
    wcat = jnp.concatenate([W[:, 0], b])
    wvec = jnp.repeat(wcat, 16)
    partials = []
    for c in range(CHUNKS):
        r0, r1 = c * CHUNK_ROWS, (c + 1) * CHUNK_ROWS
        partials.append(_sc_stage(
            trunk[r0:r1, 0], trunk[r0:r1, 1], trunk[r0:r1, 2],
            votes[r0:r1, 0], batch_idx[r0:r1], wvec))
    out = _tc_reduce(partials)
    return out.reshape(NUM_SEGMENTS, 1)


# speedup vs baseline: 1.1135x; 1.1135x over previous
import functools

import jax
import jax.numpy as jnp
from jax import lax
from jax.experimental import pallas as pl
from jax.experimental.pallas import tpu as pltpu
from jax.experimental.pallas import tpu_sc as plsc

N = 1600000
NUM_SEGMENTS = 4096
NW = 32
ROWS_PER_W = N // NW
TILE = 2000
STEPS = ROWS_PER_W // TILE
VECS = TILE // 16
UNROLL = 5

_NEG_INF = float("-inf")


def _sc_stage(t0, t1, t2, vcol, batch_idx, wvec):
    mesh = plsc.VectorSubcoreMesh(core_axis_name="c", subcore_axis_name="s")

    @functools.partial(
        pl.kernel,
        mesh=mesh,
        compiler_params=pltpu.CompilerParams(needs_layout_passes=False),
        out_type=jax.ShapeDtypeStruct((NW, NUM_SEGMENTS), jnp.float32),
        scratch_types=[
            pltpu.VMEM((TILE * 3,), jnp.float32),
            pltpu.VMEM((TILE * 3,), jnp.float32),
            pltpu.VMEM((TILE,), jnp.float32),
            pltpu.VMEM((TILE,), jnp.float32),
            pltpu.VMEM((TILE,), jnp.int32),
            pltpu.VMEM((TILE,), jnp.int32),
            pltpu.VMEM((16, NUM_SEGMENTS), jnp.float32),
            pltpu.VMEM((NUM_SEGMENTS,), jnp.float32),
            pltpu.VMEM((80,), jnp.float32),
            pltpu.SemaphoreType.DMA,
            pltpu.SemaphoreType.DMA,
        ],
    )
    def k(t0_hbm, t1_hbm, t2_hbm, votes_hbm, idx_hbm, wv_hbm, out_hbm,
          tv0, tv1, vv0, vv1, iv0, iv1, acc, red, wv_v, sem0, sem1):
        wid = lax.axis_index("s") * 2 + lax.axis_index("c")
        sems = (sem0, sem1)
        tvs, vvs, ivs = (tv0, tv1), (vv0, vv1), (iv0, iv1)

        lane = lax.iota(jnp.int32, 16)
        ninf = jnp.full((16,), _NEG_INF, jnp.float32)

        handles = [None, None]

        def fire(s):
            b = s % 2
            row0 = wid * ROWS_PER_W + s * TILE
            hs = []
            for j, col in enumerate((t0_hbm, t1_hbm, t2_hbm)):
                hs.append(pltpu.async_copy(
                    col.at[pl.ds(row0, TILE)],
                    tvs[b].at[pl.ds(j * TILE, TILE)], sems[b]))
            hs.append(pltpu.async_copy(
                votes_hbm.at[pl.ds(row0, TILE)], vvs[b], sems[b]))
            hs.append(pltpu.async_copy(
                idx_hbm.at[pl.ds(row0, TILE)], ivs[b], sems[b]))
            handles[b] = hs

        fire(0)

        pltpu.sync_copy(wv_hbm, wv_v)
        w0 = wv_v[pl.ds(0, 16)]
        w1 = wv_v[pl.ds(16, 16)]
        w2 = wv_v[pl.ds(32, 16)]
        w3 = wv_v[pl.ds(48, 16)]
        w4 = wv_v[pl.ds(64, 16)]

        def init_body(j, _):
            for r in range(16):
                acc[r, pl.ds(j * 16, 16)] = ninf
            return 0
        lax.fori_loop(0, NUM_SEGMENTS // 16, init_body, 0)

        for s in range(STEPS):
            if s + 1 < STEPS:
                fire(s + 1)
            b = s % 2
            for h in handles[b]:
                h.wait()

            def vec_body(u, _):
                xs, idxs = [], []
                for q in range(UNROLL):
                    o = u * (UNROLL * 16) + q * 16
                    c0 = tvs[b][pl.ds(o, 16)]
                    c1 = tvs[b][pl.ds(TILE + o, 16)]
                    c2 = tvs[b][pl.ds(2 * TILE + o, 16)]
                    vvv = vvs[b][pl.ds(o, 16)]
                    idxs.append(ivs[b][pl.ds(o, 16)])
                    xs.append(c0 * w0 + c1 * w1 + c2 * w2 + vvv * w3 + w4)
                for q in range(UNROLL):
                    g = plsc.load_gather(acc, [lane, idxs[q]])
                    plsc.store_scatter(acc, [lane, idxs[q]],
                                       jnp.maximum(g, xs[q]))
                return 0
            lax.fori_loop(0, VECS // UNROLL, vec_body, 0)

        def red_body(j, _):
            m = acc[0, pl.ds(j * 16, 16)]
            for r in range(1, 16):
                m = jnp.maximum(m, acc[r, pl.ds(j * 16, 16)])
            red[pl.ds(j * 16, 16)] = m
            return 0
        lax.fori_loop(0, NUM_SEGMENTS // 16, red_body, 0)

        pltpu.sync_copy(red, out_hbm.at[wid])

    return k(t0, t1, t2, vcol, batch_idx, wvec)


def _tc_reduce(partial):
    def body(p_ref, o_ref):
        o_ref[...] = jnp.max(p_ref[...], axis=0, keepdims=True)

    return pl.pallas_call(
        body,
        out_shape=jax.ShapeDtypeStruct((1, NUM_SEGMENTS), jnp.float32),
    )(partial)


def kernel(trunk, votes, batch_idx, W, b):
    wcat = jnp.concatenate([W[:, 0], b])
    wvec = jnp.repeat(wcat, 16)
    partial = _sc_stage(trunk[:, 0], trunk[:, 1], trunk[:, 2],
                        votes[:, 0], batch_idx, wvec)
    out = _tc_reduce(partial)
    return out.reshape(NUM_SEGMENTS, 1)


# speedup vs baseline: 1.5139x; 1.3596x over previous
import functools

import jax
import jax.numpy as jnp
from jax import lax
from jax.experimental import pallas as pl
from jax.experimental.pallas import tpu as pltpu
from jax.experimental.pallas import tpu_sc as plsc

N = 1600000
NUM_SEGMENTS = 4096
NW = 32
ROWS_PER_W = N // NW
TILE = 2000
STEPS = ROWS_PER_W // TILE
VECS = TILE // 16
UNROLL = 5
VWIN = 2176

_NEG_INF = float("-inf")


def _sc_stage(t0, t1, t2, vcol, batch_idx, wvec):
    mesh = plsc.VectorSubcoreMesh(core_axis_name="c", subcore_axis_name="s")

    @functools.partial(
        pl.kernel,
        mesh=mesh,
        compiler_params=pltpu.CompilerParams(needs_layout_passes=False),
        out_type=jax.ShapeDtypeStruct((NW, NUM_SEGMENTS), jnp.float32),
        scratch_types=[
            pltpu.VMEM((TILE * 3,), jnp.float32),
            pltpu.VMEM((TILE * 3,), jnp.float32),
            pltpu.VMEM((VWIN,), jnp.float32),
            pltpu.VMEM((VWIN,), jnp.float32),
            pltpu.VMEM((TILE,), jnp.int32),
            pltpu.VMEM((TILE,), jnp.int32),
            pltpu.VMEM((16, NUM_SEGMENTS), jnp.float32),
            pltpu.VMEM((NUM_SEGMENTS,), jnp.float32),
            pltpu.VMEM((80,), jnp.float32),
            pltpu.SemaphoreType.DMA,
            pltpu.SemaphoreType.DMA,
        ],
    )
    def k(t0_hbm, t1_hbm, t2_hbm, votes_hbm, idx_hbm, wv_hbm, out_hbm,
          tv0, tv1, vv0, vv1, iv0, iv1, acc, red, wv_v, sem0, sem1):
        wid = lax.axis_index("s") * 2 + lax.axis_index("c")
        sems = (sem0, sem1)
        tvs, vvs, ivs = (tv0, tv1), (vv0, vv1), (iv0, iv1)

        lane = lax.iota(jnp.int32, 16)
        ninf = jnp.full((16,), _NEG_INF, jnp.float32)

        handles = [None, None]

        def vwin_start(row0):
            a = (row0 // 128) * 128
            a = jnp.minimum(a, N - VWIN)
            return pl.multiple_of(a, 128)

        def fire(s):
            b = s % 2
            row0 = wid * ROWS_PER_W + s * TILE
            hs = []
            for j, col in enumerate((t0_hbm, t1_hbm, t2_hbm)):
                hs.append(pltpu.async_copy(
                    col.at[pl.ds(row0, TILE)],
                    tvs[b].at[pl.ds(j * TILE, TILE)], sems[b]))
            hs.append(pltpu.async_copy(
                votes_hbm.at[0, pl.ds(vwin_start(row0), VWIN)],
                vvs[b], sems[b]))
            hs.append(pltpu.async_copy(
                idx_hbm.at[pl.ds(row0, TILE)], ivs[b], sems[b]))
            handles[b] = hs

        fire(0)

        pltpu.sync_copy(wv_hbm, wv_v)
        w0 = wv_v[pl.ds(0, 16)]
        w1 = wv_v[pl.ds(16, 16)]
        w2 = wv_v[pl.ds(32, 16)]
        w3 = wv_v[pl.ds(48, 16)]
        w4 = wv_v[pl.ds(64, 16)]

        def init_body(j, _):
            for r in range(16):
                acc[r, pl.ds(j * 16, 16)] = ninf
            return 0
        lax.fori_loop(0, NUM_SEGMENTS // 16, init_body, 0)

        for s in range(STEPS):
            if s + 1 < STEPS:
                fire(s + 1)
            b = s % 2
            for h in handles[b]:
                h.wait()
            row0 = wid * ROWS_PER_W + s * TILE
            vd = row0 - vwin_start(row0)

            def vec_body(u, _):
                xs, idxs = [], []
                for q in range(UNROLL):
                    o = u * (UNROLL * 16) + q * 16
                    c0 = tvs[b][pl.ds(o, 16)]
                    c1 = tvs[b][pl.ds(TILE + o, 16)]
                    c2 = tvs[b][pl.ds(2 * TILE + o, 16)]
                    vvv = vvs[b][pl.ds(vd + o, 16)]
                    idxs.append(ivs[b][pl.ds(o, 16)])
                    xs.append(c0 * w0 + c1 * w1 + c2 * w2 + vvv * w3 + w4)
                for q in range(UNROLL):
                    g = plsc.load_gather(acc, [lane, idxs[q]])
                    plsc.store_scatter(acc, [lane, idxs[q]],
                                       jnp.maximum(g, xs[q]))
                return 0
            lax.fori_loop(0, VECS // UNROLL, vec_body, 0)

        def red_body(j, _):
            m = acc[0, pl.ds(j * 16, 16)]
            for r in range(1, 16):
                m = jnp.maximum(m, acc[r, pl.ds(j * 16, 16)])
            red[pl.ds(j * 16, 16)] = m
            return 0
        lax.fori_loop(0, NUM_SEGMENTS // 16, red_body, 0)

        pltpu.sync_copy(red, out_hbm.at[wid])

    return k(t0, t1, t2, vcol, batch_idx, wvec)


def _tc_reduce(partial):
    def body(p_ref, o_ref):
        o_ref[...] = jnp.max(p_ref[...], axis=0, keepdims=True)

    return pl.pallas_call(
        body,
        out_shape=jax.ShapeDtypeStruct((1, NUM_SEGMENTS), jnp.float32),
    )(partial)


def kernel(trunk, votes, batch_idx, W, b):
    wcat = jnp.concatenate([W[:, 0], b])
    wvec = jnp.repeat(wcat, 16)
    partial = _sc_stage(trunk[:, 0], trunk[:, 1], trunk[:, 2],
                        votes.T, batch_idx, wvec)
    out = _tc_reduce(partial)
    return out.reshape(NUM_SEGMENTS, 1)
